# baseline (speedup 1.0000x reference)
import functools

import jax
import jax.numpy as jnp
from jax import lax
from jax.experimental import pallas as pl
from jax.experimental.pallas import tpu as pltpu
from jax.experimental.pallas import tpu_sc as plsc

EMB = 64
SEQ = 200
SBLK = 256
SPLITS = ((0, 128), (128, 128))
NBUF = 2
LANES = 16


@jax.jit
def _gather_rows(pos_rows, table):
    info = plsc.get_sparse_core_info()
    nw = info.num_cores * info.num_subcores
    n_rows = pos_rows.shape[0]
    n_seq = n_rows * SBLK // SEQ
    nsb_tot = n_seq // SBLK
    per_w = n_seq // nw
    n_tab = table.shape[0]
    mesh = plsc.VectorSubcoreMesh(core_axis_name="c", subcore_axis_name="s")

    @functools.partial(
        pl.kernel,
        mesh=mesh,
        out_type=jax.ShapeDtypeStruct((SEQ, EMB, n_seq), jnp.float32),
        compiler_params=pltpu.CompilerParams(
            use_tc_tiling_on_sc=False, needs_layout_passes=False
        ),
        scratch_types=[
            pltpu.VMEM((NBUF, 1, SBLK), jnp.int32),
            pltpu.VMEM((NBUF, SBLK, EMB), jnp.float32),
            pltpu.VMEM((NBUF, EMB, SBLK), jnp.float32),
            pltpu.VMEM_SHARED((n_tab, EMB), jnp.float32),
            pltpu.SemaphoreType.DMA,
            pltpu.SemaphoreType.DMA,
            pltpu.SemaphoreType.DMA,
            pltpu.SemaphoreType.DMA,
            pltpu.SemaphoreType.DMA,
            pltpu.SemaphoreType.DMA,
        ],
    )
    def k(idx_hbm, table_hbm, out_hbm, idx_v, rows_v, trans_v, table_sp,
          si0, si1, sg0, sg1, ss0, ss1):
        wid = lax.axis_index("s") * info.num_cores + lax.axis_index("c")
        sem_i = (si0, si1)
        sem_g = (sg0, sg1)
        sem_s = (ss0, ss1)

        @pl.when(lax.axis_index("s") == 0)
        def _stage():
            pltpu.sync_copy(table_hbm, table_sp)

        plsc.subcore_barrier()

        def fire_idx(p, b):
            pj = jnp.minimum(p, SEQ - 1)
            row = pj * nsb_tot + wid * NBUF + b
            pltpu.async_copy(
                idx_hbm.at[pl.ds(row, 1), :], idx_v.at[b], sem_i[b]
            )

        def wait_idx(b):
            pltpu.make_async_copy(
                idx_hbm.at[pl.ds(0, 1), :], idx_v.at[b], sem_i[b]
            ).wait()

        def transpose(b):
            rows = rows_v.at[b]
            trans = trans_v.at[b]

            def blk(j, carry):
                q0 = j * LANES
                q_vec = q0 + lax.iota(jnp.int32, LANES)
                for e in range(EMB):
                    vals = plsc.load_gather(
                        rows, [q_vec, jnp.full((LANES,), e, jnp.int32)]
                    )
                    trans[e, pl.ds(q0, LANES)] = vals
                return carry

            lax.fori_loop(0, SBLK // LANES, blk, 0)

        def gather_and_store(p, b):
            wait_idx(b)
            handles = []
            for off, ln in SPLITS:
                handles.append(pltpu.async_copy(
                    table_sp.at[idx_v.at[b].at[0, pl.ds(off, ln)]],
                    rows_v.at[b].at[pl.ds(off, ln), :],
                    sem_g[b],
                ))
            for h in handles:
                h.wait()
            fire_idx(p + 1, b)
            transpose(b)
            pltpu.async_copy(
                trans_v.at[b],
                out_hbm.at[p].at[:, pl.ds(wid * per_w + b * SBLK, SBLK)],
                sem_s[b],
            )

        def wait_store(b):
            pltpu.make_async_copy(
                trans_v.at[b], out_hbm.at[0].at[:, pl.ds(0, SBLK)], sem_s[b]
            ).wait()

        for b in range(NBUF):
            fire_idx(0, b)
        for b in range(NBUF):
            gather_and_store(0, b)

        def body(p, carry):
            for b in range(NBUF):
                wait_store(b)
                gather_and_store(p, b)
            return carry

        lax.fori_loop(1, SEQ, body, 0)

        for b in range(NBUF):
            wait_store(b)
            wait_idx(b)

    return k(pos_rows, table)


def kernel(positions, weights):
    pos_rows = jnp.transpose(positions.astype(jnp.int32), (1, 0)).reshape(-1, SBLK)
    out = _gather_rows(pos_rows, weights)
    return lax.stop_gradient(jnp.transpose(out, (2, 0, 1)))


# speedup vs baseline: 2.1221x; 2.1221x over previous
import functools

import jax
import jax.numpy as jnp
from jax import lax
from jax.experimental import pallas as pl
from jax.experimental.pallas import tpu as pltpu
from jax.experimental.pallas import tpu_sc as plsc

EMB = 64
SEQ = 200
SBLK = 256
SPLITS = ((0, 128), (128, 128))
NBUF = 2
LANES = 16


@jax.jit
def _gather_rows(pos_rows, table):
    info = plsc.get_sparse_core_info()
    nw = info.num_cores * info.num_subcores
    n_rows = pos_rows.shape[0]
    n_seq = n_rows * SBLK // SEQ
    nsb_tot = n_seq // SBLK
    per_w = n_seq // nw
    n_tab = table.shape[0]
    mesh = plsc.VectorSubcoreMesh(core_axis_name="c", subcore_axis_name="s")

    @functools.partial(
        pl.kernel,
        mesh=mesh,
        out_type=jax.ShapeDtypeStruct((SEQ, EMB, n_seq), jnp.float32),
        compiler_params=pltpu.CompilerParams(
            use_tc_tiling_on_sc=False, needs_layout_passes=False
        ),
        scratch_types=[
            pltpu.VMEM((NBUF, 1, SBLK), jnp.int32),
            pltpu.VMEM((NBUF, SBLK, EMB), jnp.float32),
            pltpu.VMEM((NBUF, EMB, SBLK + 1), jnp.float32),
            pltpu.VMEM_SHARED((n_tab, EMB), jnp.float32),
            pltpu.SemaphoreType.DMA,
            pltpu.SemaphoreType.DMA,
            pltpu.SemaphoreType.DMA,
            pltpu.SemaphoreType.DMA,
            pltpu.SemaphoreType.DMA,
            pltpu.SemaphoreType.DMA,
        ],
    )
    def k(idx_hbm, table_hbm, out_hbm, idx_v, rows_v, trans_v, table_sp,
          si0, si1, sg0, sg1, ss0, ss1):
        wid = lax.axis_index("s") * info.num_cores + lax.axis_index("c")
        sem_i = (si0, si1)
        sem_g = (sg0, sg1)
        sem_s = (ss0, ss1)

        @pl.when(lax.axis_index("s") == 0)
        def _stage():
            pltpu.sync_copy(table_hbm, table_sp)

        plsc.subcore_barrier()

        def fire_idx(p, b):
            pj = jnp.minimum(p, SEQ - 1)
            row = pj * nsb_tot + wid * NBUF + b
            pltpu.async_copy(
                idx_hbm.at[pl.ds(row, 1), :], idx_v.at[b], sem_i[b]
            )

        def wait_idx(b):
            pltpu.make_async_copy(
                idx_hbm.at[pl.ds(0, 1), :], idx_v.at[b], sem_i[b]
            ).wait()

        def transpose(b):
            rows = rows_v.at[b]
            trans = trans_v.at[b]
            e_vecs = [g * LANES + lax.iota(jnp.int32, LANES)
                      for g in range(EMB // LANES)]

            def blk(p, carry):
                p_vec = jnp.full((LANES,), 0, jnp.int32) + p
                for g in range(EMB // LANES):
                    vals = rows[p, pl.ds(g * LANES, LANES)]
                    plsc.store_scatter(trans, [e_vecs[g], p_vec], vals)
                return carry

            lax.fori_loop(0, SBLK, blk, 0)

        def gather_and_store(p, b):
            wait_idx(b)
            handles = []
            for off, ln in SPLITS:
                handles.append(pltpu.async_copy(
                    table_sp.at[idx_v.at[b].at[0, pl.ds(off, ln)]],
                    rows_v.at[b].at[pl.ds(off, ln), :],
                    sem_g[b],
                ))
            for h in handles:
                h.wait()
            fire_idx(p + 1, b)
            transpose(b)
            pltpu.async_copy(
                trans_v.at[b].at[:, pl.ds(0, SBLK)],
                out_hbm.at[p].at[:, pl.ds(wid * per_w + b * SBLK, SBLK)],
                sem_s[b],
            )

        def wait_store(b):
            pltpu.make_async_copy(
                trans_v.at[b].at[:, pl.ds(0, SBLK)],
                out_hbm.at[0].at[:, pl.ds(0, SBLK)], sem_s[b]
            ).wait()

        for b in range(NBUF):
            fire_idx(0, b)
        for b in range(NBUF):
            gather_and_store(0, b)

        def body(p, carry):
            for b in range(NBUF):
                wait_store(b)
                gather_and_store(p, b)
            return carry

        lax.fori_loop(1, SEQ, body, 0)

        for b in range(NBUF):
            wait_store(b)
            wait_idx(b)

    return k(pos_rows, table)


def kernel(positions, weights):
    pos_rows = jnp.transpose(positions.astype(jnp.int32), (1, 0)).reshape(-1, SBLK)
    out = _gather_rows(pos_rows, weights)
    return lax.stop_gradient(jnp.transpose(out, (2, 0, 1)))


# speedup vs baseline: 2.2052x; 1.0391x over previous
import functools

import jax
import jax.numpy as jnp
from jax import lax
from jax.experimental import pallas as pl
from jax.experimental.pallas import tpu as pltpu
from jax.experimental.pallas import tpu_sc as plsc

EMB = 64
SEQ = 200
SBLK = 256
SPLITS = ((0, 128), (128, 128))
NBUF = 2
LANES = 16


@jax.jit
def _gather_rows(pos_rows, table):
    info = plsc.get_sparse_core_info()
    nw = info.num_cores * info.num_subcores
    n_rows = pos_rows.shape[0]
    n_seq = n_rows * SBLK // SEQ
    nsb_tot = n_seq // SBLK
    per_w = n_seq // nw
    n_tab = table.shape[0]
    mesh = plsc.VectorSubcoreMesh(core_axis_name="c", subcore_axis_name="s")

    @functools.partial(
        pl.kernel,
        mesh=mesh,
        out_type=jax.ShapeDtypeStruct((SEQ, EMB, n_seq), jnp.float32),
        compiler_params=pltpu.CompilerParams(
            use_tc_tiling_on_sc=False, needs_layout_passes=False
        ),
        scratch_types=[
            pltpu.VMEM((NBUF, 1, SBLK), jnp.int32),
            pltpu.VMEM((NBUF, SBLK, EMB), jnp.float32),
            pltpu.VMEM((NBUF, EMB, SBLK + 1), jnp.float32),
            pltpu.VMEM_SHARED((n_tab, EMB), jnp.float32),
            pltpu.SemaphoreType.DMA,
            pltpu.SemaphoreType.DMA,
            pltpu.SemaphoreType.DMA,
            pltpu.SemaphoreType.DMA,
            pltpu.SemaphoreType.DMA,
            pltpu.SemaphoreType.DMA,
        ],
    )
    def k(idx_hbm, table_hbm, out_hbm, idx_v, rows_v, trans_v, table_sp,
          si0, si1, sg0, sg1, ss0, ss1):
        wid = lax.axis_index("s") * info.num_cores + lax.axis_index("c")
        sem_i = (si0, si1)
        sem_g = (sg0, sg1)
        sem_s = (ss0, ss1)

        @pl.when(lax.axis_index("s") == 0)
        def _stage():
            pltpu.sync_copy(table_hbm, table_sp)

        plsc.subcore_barrier()

        def fire_idx(p, b):
            pj = jnp.minimum(p, SEQ - 1)
            row = pj * nsb_tot + wid * NBUF + b
            pltpu.async_copy(
                idx_hbm.at[pl.ds(row, 1), :], idx_v.at[b], sem_i[b]
            )

        def wait_idx(b):
            pltpu.make_async_copy(
                idx_hbm.at[pl.ds(0, 1), :], idx_v.at[b], sem_i[b]
            ).wait()

        def transpose(b):
            rows = rows_v.at[b]
            trans = trans_v.at[b]
            e_vecs = [g * LANES + lax.iota(jnp.int32, LANES)
                      for g in range(EMB // LANES)]

            def blk(j, carry):
                p0 = j * 8
                for dp in range(8):
                    p = p0 + dp
                    p_vec = jnp.full((LANES,), 0, jnp.int32) + p
                    for g in range(EMB // LANES):
                        vals = rows[p, pl.ds(g * LANES, LANES)]
                        plsc.store_scatter(trans, [e_vecs[g], p_vec], vals)
                return carry

            lax.fori_loop(0, SBLK // 8, blk, 0)

        def gather_and_store(p, b):
            wait_idx(b)
            handles = []
            for off, ln in SPLITS:
                handles.append(pltpu.async_copy(
                    table_sp.at[idx_v.at[b].at[0, pl.ds(off, ln)]],
                    rows_v.at[b].at[pl.ds(off, ln), :],
                    sem_g[b],
                ))
            for h in handles:
                h.wait()
            fire_idx(p + 1, b)
            transpose(b)
            pltpu.async_copy(
                trans_v.at[b].at[:, pl.ds(0, SBLK)],
                out_hbm.at[p].at[:, pl.ds(wid * per_w + b * SBLK, SBLK)],
                sem_s[b],
            )

        def wait_store(b):
            pltpu.make_async_copy(
                trans_v.at[b].at[:, pl.ds(0, SBLK)],
                out_hbm.at[0].at[:, pl.ds(0, SBLK)], sem_s[b]
            ).wait()

        for b in range(NBUF):
            fire_idx(0, b)
        for b in range(NBUF):
            gather_and_store(0, b)

        def body(p, carry):
            for b in range(NBUF):
                wait_store(b)
                gather_and_store(p, b)
            return carry

        lax.fori_loop(1, SEQ, body, 0)

        for b in range(NBUF):
            wait_store(b)
            wait_idx(b)

    return k(pos_rows, table)


def kernel(positions, weights):
    pos_rows = jnp.transpose(positions.astype(jnp.int32), (1, 0)).reshape(-1, SBLK)
    out = _gather_rows(pos_rows, weights)
    return lax.stop_gradient(jnp.transpose(out, (2, 0, 1)))


# speedup vs baseline: 2.6255x; 1.1906x over previous
import functools

import jax
import jax.numpy as jnp
from jax import lax
from jax.experimental import pallas as pl
from jax.experimental.pallas import tpu as pltpu
from jax.experimental.pallas import tpu_sc as plsc

EMB = 64
SEQ = 200
CSEQ = 2
SPLITS = ((0, 128), (128, 72))
NBUF = 2


@jax.jit
def _gather_rows(positions, table):
    info = plsc.get_sparse_core_info()
    nw = info.num_cores * info.num_subcores
    n_seq, seq = positions.shape
    per_w = n_seq // nw
    n_chunks = per_w // CSEQ
    n_outer = n_chunks // NBUF
    n_tab = table.shape[0]
    mesh = plsc.VectorSubcoreMesh(core_axis_name="c", subcore_axis_name="s")

    @functools.partial(
        pl.kernel,
        mesh=mesh,
        out_type=jax.ShapeDtypeStruct((n_seq, seq, EMB), jnp.float32),
        compiler_params=pltpu.CompilerParams(use_tc_tiling_on_sc=False),
        scratch_types=[
            pltpu.VMEM((NBUF, CSEQ, SEQ), jnp.int32),
            pltpu.VMEM((NBUF, CSEQ, SEQ, EMB), jnp.float32),
            pltpu.VMEM_SHARED((n_tab, EMB), jnp.float32),
            pltpu.SemaphoreType.DMA,
            pltpu.SemaphoreType.DMA,
            pltpu.SemaphoreType.DMA,
            pltpu.SemaphoreType.DMA,
            pltpu.SemaphoreType.DMA,
            pltpu.SemaphoreType.DMA,
        ],
    )
    def k(idx_hbm, table_hbm, out_hbm, idx_v, rows_v, table_sp,
          si0, si1, sg0, sg1, ss0, ss1):
        wid = lax.axis_index("s") * info.num_cores + lax.axis_index("c")
        seq0 = wid * per_w
        sem_i = (si0, si1)
        sem_g = (sg0, sg1)
        sem_s = (ss0, ss1)

        @pl.when(lax.axis_index("s") == 0)
        def _stage():
            pltpu.sync_copy(table_hbm, table_sp)

        plsc.subcore_barrier()

        def fire_idx(ci, b):
            cj = jnp.minimum(ci, n_chunks - 1)
            pltpu.async_copy(
                idx_hbm.at[pl.ds(seq0 + cj * CSEQ, CSEQ), :], idx_v.at[b], sem_i[b]
            )

        def wait_idx(b):
            pltpu.make_async_copy(
                idx_hbm.at[pl.ds(0, CSEQ), :], idx_v.at[b], sem_i[b]
            ).wait()

        def gather_and_store(ci, b):
            wait_idx(b)
            handles = []
            for s in range(CSEQ):
                for off, ln in SPLITS:
                    handles.append(pltpu.async_copy(
                        table_sp.at[idx_v.at[b].at[s, pl.ds(off, ln)]],
                        rows_v.at[b].at[s, pl.ds(off, ln), :],
                        sem_g[b],
                    ))
            for h in handles:
                h.wait()
            fire_idx(ci + NBUF, b)
            pltpu.async_copy(
                rows_v.at[b], out_hbm.at[pl.ds(seq0 + ci * CSEQ, CSEQ), :, :], sem_s[b]
            )

        def wait_store(b):
            pltpu.make_async_copy(
                rows_v.at[b], out_hbm.at[pl.ds(0, CSEQ), :, :], sem_s[b]
            ).wait()

        for b in range(NBUF):
            fire_idx(b, b)
        for b in range(NBUF):
            gather_and_store(b, b)

        def body(g, carry):
            for b in range(NBUF):
                ci = g * NBUF + b
                wait_store(b)
                gather_and_store(ci, b)
            return carry

        lax.fori_loop(1, n_outer, body, 0)

        for b in range(NBUF):
            wait_store(b)
            wait_idx(b)

    return k(positions, table)


def kernel(positions, weights):
    out = _gather_rows(positions.astype(jnp.int32), weights)
    return lax.stop_gradient(out)
